# baseline (speedup 1.0000x reference)
import functools

import jax
import jax.numpy as jnp
from jax import lax
from jax.experimental import pallas as pl
from jax.experimental.pallas import tpu as pltpu
from jax.experimental.pallas import tpu_sc as plsc

_B, _C, _H, _W = 32, 384, 64, 64
_HW = _H * _W
_NC, _NS = 2, 16


@functools.partial(
    pl.kernel,
    out_type=jax.ShapeDtypeStruct((_C,), jnp.float32),
    mesh=plsc.VectorSubcoreMesh(
        core_axis_name="c", subcore_axis_name="s",
        num_cores=_NC, num_subcores=_NS,
    ),
    scratch_types=[
        pltpu.VMEM((16,), jnp.int32),
        pltpu.VMEM((8 * _C,), jnp.float32),
    ],
)
def _sc_gather(st_hbm, e_hbm, o_hbm, st_s, emb_v):
    wid = lax.axis_index("s") * _NC + lax.axis_index("c")

    @pl.when(wid == 0)
    def _():
        pltpu.sync_copy(st_hbm, st_s)
        pltpu.sync_copy(e_hbm, emb_v)
        st = st_s[...][0]
        pltpu.sync_copy(emb_v.at[pl.ds(st * _C, _C)], o_hbm)


def _tc_body(x_ref, e_ref, o_ref):
    o_ref[...] = x_ref[...] * e_ref[...]


def _tc_scale(x3, scale3):
    return pl.pallas_call(
        _tc_body,
        grid=(_B,),
        in_specs=[
            pl.BlockSpec((1, _C, _HW), lambda i: (i, 0, 0)),
            pl.BlockSpec((1, _C, 1), lambda i: (0, 0, 0)),
        ],
        out_specs=pl.BlockSpec((1, _C, _HW), lambda i: (i, 0, 0)),
        out_shape=jax.ShapeDtypeStruct((_B, _C, _HW), jnp.float32),
        compiler_params=pltpu.CompilerParams(
            dimension_semantics=("arbitrary",),
            needs_layout_passes=True,
        ),
    )(x3, scale3)


def kernel(x, stage, embeddings):
    s = jnp.full((16,), stage, dtype=jnp.int32)
    scale = _sc_gather(s, embeddings.reshape(-1))
    out = _tc_scale(x.reshape(_B, _C, _HW), scale.reshape(1, _C, 1))
    return out.reshape(_B, _C, _H, _W)
